# baseline (speedup 1.0000x reference)
import functools

import jax
import jax.numpy as jnp
from jax import lax
from jax.experimental import pallas as pl
from jax.experimental.pallas import tpu as pltpu
from jax.experimental.pallas import tpu_sc as plsc

N_NODES = 10000
DIM = 128
E = 160000

NC = 2
NS = 16
NW = NC * NS
CHUNK = 128
EPAD = 163840
CPT = EPAD // (NW * CHUNK)
NACC = 10112
STRIPE = NACC // NS


NBUF = 2


@functools.cache
def _make_segsum():
    mesh = plsc.VectorSubcoreMesh(core_axis_name="c", subcore_axis_name="s",
                                  num_cores=NC, num_subcores=NS)

    HCH = CHUNK // 2

    def body(table, src, dst, zeros, out, acc, src_v, dst_v, *bufs):
        rows = bufs[:NBUF]
        gsem = bufs[NBUF:3 * NBUF]
        ssem = bufs[3 * NBUF:4 * NBUF]
        c = lax.axis_index("c")
        s = lax.axis_index("s")
        w = s * NC + c
        pltpu.sync_copy(src.at[pl.ds(w * CPT, CPT)], src_v)
        pltpu.sync_copy(dst.at[pl.ds(w * CPT, CPT)], dst_v)
        pltpu.sync_copy(zeros.at[pl.ds(s * STRIPE, STRIPE)],
                        acc.at[pl.ds(s * STRIPE, STRIPE)])

        def start_gather(i, b):
            return [
                pltpu.async_copy(table.at[src_v.at[i, pl.ds(h * HCH, HCH)]],
                                 rows[b].at[pl.ds(h * HCH, HCH)],
                                 gsem[2 * b + h])
                for h in range(2)]

        gd = [start_gather(b, b) for b in range(NBUF)]
        plsc.subcore_barrier()

        def group(g, carry):
            sd = []
            for b in range(NBUF):
                gd[b][0].wait()
                gd[b][1].wait()
                sd.append(pltpu.async_copy(
                    rows[b], acc.at[dst_v.at[g * NBUF + b]], ssem[b],
                    add=True))
            for b in range(NBUF):
                sd[b].wait()
                start_gather((g + 1) * NBUF + b, b)
            return carry

        lax.fori_loop(0, CPT // NBUF - 1, group, 0)
        g_last = CPT // NBUF - 1
        sd = []
        for b in range(NBUF):
            gd[b][0].wait()
            gd[b][1].wait()
            sd.append(pltpu.async_copy(
                rows[b], acc.at[dst_v.at[g_last * NBUF + b]], ssem[b],
                add=True))
        for b in range(NBUF):
            sd[b].wait()
        plsc.subcore_barrier()
        pltpu.sync_copy(acc.at[pl.ds(s * STRIPE, STRIPE)],
                        out.at[c, pl.ds(s * STRIPE, STRIPE)])

    return pl.kernel(
        body,
        out_type=jax.ShapeDtypeStruct((NC, NACC, DIM), jnp.float32),
        mesh=mesh,
        scratch_types=[
            pltpu.VMEM_SHARED((NACC, DIM), jnp.float32),
            pltpu.VMEM((CPT, CHUNK), jnp.int32),
            pltpu.VMEM((CPT, CHUNK), jnp.int32),
        ] + [pltpu.VMEM((CHUNK, DIM), jnp.float32) for _ in range(NBUF)]
          + [pltpu.SemaphoreType.DMA for _ in range(3 * NBUF)])


RTAB = 512


@functools.cache
def _make_segsum_sptab():
    mesh = plsc.VectorSubcoreMesh(core_axis_name="c", subcore_axis_name="s",
                                  num_cores=NC, num_subcores=NS)
    TSTRIPE = RTAB // NS

    def body(table, src, dst, zeros, out, out_deg,
             acc, rtab, src_v, dst_v, rows, deg_v, gsem, ssem):
        c = lax.axis_index("c")
        s = lax.axis_index("s")
        w = s * NC + c
        pltpu.sync_copy(src.at[pl.ds(w * CPT, CPT)], src_v)
        pltpu.sync_copy(dst.at[pl.ds(w * CPT, CPT)], dst_v)
        pltpu.sync_copy(table.at[pl.ds(s * TSTRIPE, TSTRIPE)],
                        rtab.at[pl.ds(s * TSTRIPE, TSTRIPE)])
        pltpu.sync_copy(zeros.at[pl.ds(s * STRIPE, STRIPE)],
                        acc.at[pl.ds(s * STRIPE, STRIPE)])
        z16 = jnp.zeros((16,), jnp.float32)

        def zero_deg(i, carry):
            deg_v[pl.ds(i * 16, 16)] = z16
            return carry

        lax.fori_loop(0, NACC // 16, zero_deg, 0)
        plsc.subcore_barrier()
        gd = pltpu.async_copy(rtab.at[src_v.at[0]], rows, gsem)
        ones16 = jnp.ones((16,), jnp.float32)

        def chunk(i, carry):
            gd.wait()
            sd = pltpu.async_copy(rows, acc.at[dst_v.at[i]], ssem, add=True)
            dst_row = dst_v.at[i]
            for j in range(CHUNK // 16):
                idx16 = dst_row[pl.ds(j * 16, 16)]
                plsc.addupdate_scatter(deg_v, [idx16], ones16)
            sd.wait()
            nxt = jnp.minimum(i + 1, CPT - 1)
            pltpu.async_copy(rtab.at[src_v.at[nxt]], rows, gsem)
            return carry

        lax.fori_loop(0, CPT, chunk, 0)
        gd.wait()
        plsc.subcore_barrier()
        pltpu.sync_copy(acc.at[pl.ds(s * STRIPE, STRIPE)],
                        out.at[c, pl.ds(s * STRIPE, STRIPE)])
        pltpu.sync_copy(deg_v, out_deg.at[c, s])

    return pl.kernel(
        body,
        out_type=(jax.ShapeDtypeStruct((NC, NACC, DIM), jnp.float32),
                  jax.ShapeDtypeStruct((NC, NS, NACC), jnp.float32)),
        mesh=mesh,
        compiler_params=pltpu.CompilerParams(needs_layout_passes=False),
        scratch_types=[
            pltpu.VMEM_SHARED((NACC, DIM), jnp.float32),
            pltpu.VMEM_SHARED((RTAB, DIM), jnp.float32),
            pltpu.VMEM((CPT, CHUNK), jnp.int32),
            pltpu.VMEM((CPT, CHUNK), jnp.int32),
            pltpu.VMEM((CHUNK, DIM), jnp.float32),
            pltpu.VMEM((NACC,), jnp.float32),
            pltpu.SemaphoreType.DMA,
            pltpu.SemaphoreType.DMA,
        ])


ROWB = 1000


def _layer_body(a_ref, s_ref, d_ref, cur_ref, wn_ref, ws_ref, out_ref):
    a = a_ref[0] + a_ref[1] + s_ref[0] + s_ref[1]
    deg = jnp.sum(d_ref[...], axis=1, keepdims=True)
    agg = a / jnp.maximum(deg, 1.0)
    out_ref[...] = (
        jnp.dot(agg, wn_ref[...], preferred_element_type=jnp.float32)
        + jnp.dot(cur_ref[...], ws_ref[...], preferred_element_type=jnp.float32))


def _layer_dense(a, sagg, deg, cur, wn, ws):
    return pl.pallas_call(
        _layer_body,
        grid=(N_NODES // ROWB,),
        in_specs=[
            pl.BlockSpec((NC, ROWB, DIM), lambda i: (0, i, 0)),
            pl.BlockSpec((NC, ROWB, DIM), lambda i: (0, i, 0)),
            pl.BlockSpec((ROWB, NW), lambda i: (i, 0)),
            pl.BlockSpec((ROWB, DIM), lambda i: (i, 0)),
            pl.BlockSpec((DIM, DIM), lambda i: (0, 0)),
            pl.BlockSpec((DIM, DIM), lambda i: (0, 0)),
        ],
        out_specs=pl.BlockSpec((ROWB, DIM), lambda i: (i, 0)),
        out_shape=jax.ShapeDtypeStruct((N_NODES, DIM), jnp.float32),
    )(a, sagg, deg, cur, wn, ws)


def _l2n(x):
    n = jnp.sqrt(jnp.sum(x * x, axis=1, keepdims=True))
    return x / jnp.maximum(n, 1e-12)


def _gate_body(h_ref, cur_ref, tg_ref, tb_ref, out_ref):
    h = h_ref[...]
    curn = _l2n(cur_ref[...])
    tw = jax.nn.sigmoid(
        jnp.dot(h, tg_ref[...], preferred_element_type=jnp.float32)
        + tb_ref[...])
    out_ref[...] = _l2n(tw * curn + (1.0 - tw) * h)


def _gate_dense(h, cur, tg, tb):
    return pl.pallas_call(
        _gate_body,
        grid=(N_NODES // ROWB,),
        in_specs=[
            pl.BlockSpec((ROWB, DIM), lambda i: (i, 0)),
            pl.BlockSpec((ROWB, DIM), lambda i: (i, 0)),
            pl.BlockSpec((DIM, DIM), lambda i: (0, 0)),
            pl.BlockSpec((1, DIM), lambda i: (0, 0)),
        ],
        out_specs=pl.BlockSpec((ROWB, DIM), lambda i: (i, 0)),
        out_shape=jax.ShapeDtypeStruct((N_NODES, DIM), jnp.float32),
    )(h, cur, tg, tb.reshape(1, DIM))


def _l2n_body(x_ref, o_ref):
    o_ref[...] = _l2n(x_ref[...])


def _l2norm_pallas(x):
    return pl.pallas_call(
        _l2n_body,
        out_shape=jax.ShapeDtypeStruct(x.shape, jnp.float32),
    )(x)


def _prep_idx(idx, pad_val):
    pad = jnp.full((EPAD - E,), pad_val, jnp.int32)
    return jnp.concatenate([idx, pad]).reshape(EPAD // CHUNK, CHUNK)


def kernel(entity_embed, relation_embed, edges_src, edges_rel, edges_dst,
           time_gate_weight, time_gate_bias, W_neigh, W_self):
    zeros = jnp.zeros((NACC, DIM), jnp.float32)
    ones = jnp.ones((CHUNK, DIM), jnp.float32)

    h = _l2norm_pallas(entity_embed)
    r = _l2norm_pallas(relation_embed)
    r_pad = jnp.pad(r, ((0, RTAB - r.shape[0]), (0, 0)))

    T = edges_src.shape[0]
    for t in range(T):
        src = _prep_idx(edges_src[t], 0)
        rel = _prep_idx(edges_rel[t], 0)
        dst = _prep_idx(edges_dst[t], N_NODES)
        sagg, degp = _make_segsum_sptab()(r_pad, rel, dst, zeros)
        deg = jnp.transpose(degp.reshape(NW, NACC))
        cur = h
        for l in range(W_neigh.shape[0]):
            a = _make_segsum()(cur, src, dst, zeros)
            cur = _layer_dense(a, sagg, deg, cur, W_neigh[l], W_self[l])
        h = _gate_dense(h, cur, time_gate_weight, time_gate_bias)
    return h


# speedup vs baseline: 1.9214x; 1.9214x over previous
import functools

import jax
import jax.numpy as jnp
from jax import lax
from jax.experimental import pallas as pl
from jax.experimental.pallas import tpu as pltpu
from jax.experimental.pallas import tpu_sc as plsc

N_NODES = 10000
DIM = 128
E = 160000

NC = 2
NS = 16
NW = NC * NS
CHUNK = 128
EPAD = 163840
CPT = EPAD // (NW * CHUNK)
NACC = 10112
STRIPE = NACC // NS


NBUF = 2


@functools.cache
def _make_segsum():
    mesh = plsc.VectorSubcoreMesh(core_axis_name="c", subcore_axis_name="s",
                                  num_cores=NC, num_subcores=NS)

    HCH = CHUNK // 2

    def body(table, src, dst, zeros, out, acc, src_v, dst_v, *bufs):
        rows = bufs[:NBUF]
        gsem = bufs[NBUF:3 * NBUF]
        ssem = bufs[3 * NBUF:4 * NBUF]
        c = lax.axis_index("c")
        s = lax.axis_index("s")
        w = s * NC + c
        pltpu.sync_copy(src.at[pl.ds(w * CPT, CPT)], src_v)
        pltpu.sync_copy(dst.at[pl.ds(w * CPT, CPT)], dst_v)
        pltpu.sync_copy(zeros.at[pl.ds(s * STRIPE, STRIPE)],
                        acc.at[pl.ds(s * STRIPE, STRIPE)])

        def start_gather(i, b):
            return [
                pltpu.async_copy(table.at[src_v.at[i, pl.ds(h * HCH, HCH)]],
                                 rows[b].at[pl.ds(h * HCH, HCH)],
                                 gsem[2 * b + h])
                for h in range(2)]

        gd = [start_gather(b, b) for b in range(NBUF)]
        plsc.subcore_barrier()

        def group(g, carry):
            sd = []
            for b in range(NBUF):
                gd[b][0].wait()
                gd[b][1].wait()
                sd.append(pltpu.async_copy(
                    rows[b], acc.at[dst_v.at[g * NBUF + b]], ssem[b],
                    add=True))
            for b in range(NBUF):
                sd[b].wait()
                start_gather((g + 1) * NBUF + b, b)
            return carry

        lax.fori_loop(0, CPT // NBUF - 1, group, 0)
        g_last = CPT // NBUF - 1
        sd = []
        for b in range(NBUF):
            gd[b][0].wait()
            gd[b][1].wait()
            sd.append(pltpu.async_copy(
                rows[b], acc.at[dst_v.at[g_last * NBUF + b]], ssem[b],
                add=True))
        for b in range(NBUF):
            sd[b].wait()
        plsc.subcore_barrier()
        pltpu.sync_copy(acc.at[pl.ds(s * STRIPE, STRIPE)],
                        out.at[c, pl.ds(s * STRIPE, STRIPE)])

    return pl.kernel(
        body,
        out_type=jax.ShapeDtypeStruct((NC, NACC, DIM), jnp.float32),
        mesh=mesh,
        scratch_types=[
            pltpu.VMEM_SHARED((NACC, DIM), jnp.float32),
            pltpu.VMEM((CPT, CHUNK), jnp.int32),
            pltpu.VMEM((CPT, CHUNK), jnp.int32),
        ] + [pltpu.VMEM((CHUNK, DIM), jnp.float32) for _ in range(NBUF)]
          + [pltpu.SemaphoreType.DMA for _ in range(3 * NBUF)])


RTAB = 512


@functools.cache
def _make_segsum_sptab():
    mesh = plsc.VectorSubcoreMesh(core_axis_name="c", subcore_axis_name="s",
                                  num_cores=NC, num_subcores=NS)
    TSTRIPE = RTAB // NS

    def body(table, src, dst, zeros, out, acc, rtab, src_v, dst_v, *bufs):
        rows = bufs[:NBUF]
        gsem = bufs[NBUF:2 * NBUF]
        ssem = bufs[2 * NBUF:3 * NBUF]
        c = lax.axis_index("c")
        s = lax.axis_index("s")
        w = s * NC + c
        pltpu.sync_copy(src.at[pl.ds(w * CPT, CPT)], src_v)
        pltpu.sync_copy(dst.at[pl.ds(w * CPT, CPT)], dst_v)
        pltpu.sync_copy(table.at[pl.ds(s * TSTRIPE, TSTRIPE)],
                        rtab.at[pl.ds(s * TSTRIPE, TSTRIPE)])
        pltpu.sync_copy(zeros.at[pl.ds(s * STRIPE, STRIPE)],
                        acc.at[pl.ds(s * STRIPE, STRIPE)])
        plsc.subcore_barrier()
        gd = [pltpu.async_copy(rtab.at[src_v.at[b]], rows[b], gsem[b])
              for b in range(NBUF)]

        def group(g, carry):
            sd = []
            for b in range(NBUF):
                gd[b].wait()
                sd.append(pltpu.async_copy(
                    rows[b], acc.at[dst_v.at[g * NBUF + b]], ssem[b],
                    add=True))
            for b in range(NBUF):
                sd[b].wait()
                pltpu.async_copy(rtab.at[src_v.at[(g + 1) * NBUF + b]],
                                 rows[b], gsem[b])
            return carry

        lax.fori_loop(0, CPT // NBUF - 1, group, 0)
        g_last = CPT // NBUF - 1
        sd = []
        for b in range(NBUF):
            gd[b].wait()
            sd.append(pltpu.async_copy(
                rows[b], acc.at[dst_v.at[g_last * NBUF + b]], ssem[b],
                add=True))
        for b in range(NBUF):
            sd[b].wait()
        plsc.subcore_barrier()
        pltpu.sync_copy(acc.at[pl.ds(s * STRIPE, STRIPE)],
                        out.at[c, pl.ds(s * STRIPE, STRIPE)])

    return pl.kernel(
        body,
        out_type=jax.ShapeDtypeStruct((NC, NACC, DIM), jnp.float32),
        mesh=mesh,
        scratch_types=[
            pltpu.VMEM_SHARED((NACC, DIM), jnp.float32),
            pltpu.VMEM_SHARED((RTAB, DIM), jnp.float32),
            pltpu.VMEM((CPT, CHUNK), jnp.int32),
            pltpu.VMEM((CPT, CHUNK), jnp.int32),
        ] + [pltpu.VMEM((CHUNK, DIM), jnp.float32) for _ in range(NBUF)]
          + [pltpu.SemaphoreType.DMA for _ in range(2 * NBUF)])


@functools.cache
def _make_degcount():
    mesh = plsc.VectorSubcoreMesh(core_axis_name="c", subcore_axis_name="s",
                                  num_cores=NC, num_subcores=NS)

    DGRP = 8

    def body(dst, zeros, ones, out, acc, dst_v, ones_v, *sems):
        c = lax.axis_index("c")
        s = lax.axis_index("s")
        w = s * NC + c
        pltpu.sync_copy(dst.at[pl.ds(w * CPT, CPT)], dst_v)
        pltpu.sync_copy(zeros.at[pl.ds(s * STRIPE, STRIPE)],
                        acc.at[pl.ds(s * STRIPE, STRIPE)])
        pltpu.sync_copy(ones, ones_v)
        plsc.subcore_barrier()

        def group(g, carry):
            sd = [pltpu.async_copy(ones_v, acc.at[dst_v.at[g * DGRP + b]],
                                   sems[b], add=True)
                  for b in range(DGRP)]
            for d in sd:
                d.wait()
            return carry

        lax.fori_loop(0, CPT // DGRP, group, 0)
        plsc.subcore_barrier()
        pltpu.sync_copy(acc.at[pl.ds(s * STRIPE, STRIPE)],
                        out.at[c, pl.ds(s * STRIPE, STRIPE)])

    return pl.kernel(
        body,
        out_type=jax.ShapeDtypeStruct((NC, NACC, DIM), jnp.float32),
        mesh=mesh,
        scratch_types=[
            pltpu.VMEM_SHARED((NACC, DIM), jnp.float32),
            pltpu.VMEM((CPT, CHUNK), jnp.int32),
            pltpu.VMEM((CHUNK, DIM), jnp.float32),
        ] + [pltpu.SemaphoreType.DMA for _ in range(DGRP)])


ROWB = 1000


def _layer_body(a_ref, s_ref, d_ref, cur_ref, wn_ref, ws_ref, out_ref):
    a = a_ref[0] + a_ref[1] + s_ref[0] + s_ref[1]
    deg = d_ref[0, :, 0:1] + d_ref[1, :, 0:1]
    agg = a / jnp.maximum(deg, 1.0)
    out_ref[...] = (
        jnp.dot(agg, wn_ref[...], preferred_element_type=jnp.float32)
        + jnp.dot(cur_ref[...], ws_ref[...], preferred_element_type=jnp.float32))


def _layer_dense(a, sagg, deg, cur, wn, ws):
    return pl.pallas_call(
        _layer_body,
        grid=(N_NODES // ROWB,),
        in_specs=[
            pl.BlockSpec((NC, ROWB, DIM), lambda i: (0, i, 0)),
            pl.BlockSpec((NC, ROWB, DIM), lambda i: (0, i, 0)),
            pl.BlockSpec((NC, ROWB, DIM), lambda i: (0, i, 0)),
            pl.BlockSpec((ROWB, DIM), lambda i: (i, 0)),
            pl.BlockSpec((DIM, DIM), lambda i: (0, 0)),
            pl.BlockSpec((DIM, DIM), lambda i: (0, 0)),
        ],
        out_specs=pl.BlockSpec((ROWB, DIM), lambda i: (i, 0)),
        out_shape=jax.ShapeDtypeStruct((N_NODES, DIM), jnp.float32),
    )(a, sagg, deg, cur, wn, ws)


def _l2n(x):
    n = jnp.sqrt(jnp.sum(x * x, axis=1, keepdims=True))
    return x / jnp.maximum(n, 1e-12)


def _gate_body(h_ref, cur_ref, tg_ref, tb_ref, out_ref):
    h = h_ref[...]
    curn = _l2n(cur_ref[...])
    tw = jax.nn.sigmoid(
        jnp.dot(h, tg_ref[...], preferred_element_type=jnp.float32)
        + tb_ref[...])
    out_ref[...] = _l2n(tw * curn + (1.0 - tw) * h)


def _gate_dense(h, cur, tg, tb):
    return pl.pallas_call(
        _gate_body,
        grid=(N_NODES // ROWB,),
        in_specs=[
            pl.BlockSpec((ROWB, DIM), lambda i: (i, 0)),
            pl.BlockSpec((ROWB, DIM), lambda i: (i, 0)),
            pl.BlockSpec((DIM, DIM), lambda i: (0, 0)),
            pl.BlockSpec((1, DIM), lambda i: (0, 0)),
        ],
        out_specs=pl.BlockSpec((ROWB, DIM), lambda i: (i, 0)),
        out_shape=jax.ShapeDtypeStruct((N_NODES, DIM), jnp.float32),
    )(h, cur, tg, tb.reshape(1, DIM))


def _l2n_body(x_ref, o_ref):
    o_ref[...] = _l2n(x_ref[...])


def _l2norm_pallas(x):
    return pl.pallas_call(
        _l2n_body,
        out_shape=jax.ShapeDtypeStruct(x.shape, jnp.float32),
    )(x)


def _prep_idx(idx, pad_base, pad_mod):
    pad = pad_base + (jnp.arange(EPAD - E, dtype=jnp.int32) % pad_mod)
    return jnp.concatenate([idx, pad]).reshape(EPAD // CHUNK, CHUNK)


def kernel(entity_embed, relation_embed, edges_src, edges_rel, edges_dst,
           time_gate_weight, time_gate_bias, W_neigh, W_self):
    zeros = jnp.zeros((NACC, DIM), jnp.float32)
    ones = jnp.ones((CHUNK, DIM), jnp.float32)

    h = _l2norm_pallas(entity_embed)
    r = _l2norm_pallas(relation_embed)
    r_pad = jnp.pad(r, ((0, RTAB - r.shape[0]), (0, 0)))

    T = edges_src.shape[0]
    for t in range(T):
        src = _prep_idx(edges_src[t], 0, N_NODES)
        rel = _prep_idx(edges_rel[t], 0, 500)
        dst = _prep_idx(edges_dst[t], N_NODES, NACC - N_NODES)
        sagg = _make_segsum_sptab()(r_pad, rel, dst, zeros)
        deg = _make_degcount()(dst, zeros, ones)
        cur = h
        for l in range(W_neigh.shape[0]):
            a = _make_segsum()(cur, src, dst, zeros)
            cur = _layer_dense(a, sagg, deg, cur, W_neigh[l], W_self[l])
        h = _gate_dense(h, cur, time_gate_weight, time_gate_bias)
    return h
